# baseline (speedup 1.0000x reference)
import dataclasses
import functools

import jax
import jax.numpy as jnp
from jax import lax
from jax.experimental import pallas as pl
from jax.experimental.pallas import tpu as pltpu
from jax.experimental.pallas import tpu_sc as plsc

_N = 10000
_E = 160000
_D_IN = 256
_D_H = 256
_N_CLS = 128

_NT = 16
_NC = 2
_CHUNK = 128
_NBUF = 2
_EPT = _E // _NT
_EPT_PAD = 10240
_CH = _EPT_PAD // _CHUNK
_CHD = _EPT_PAD // _CHUNK
_EPT2 = _E // (_NC * _NT)
_EPT2_PAD = 5120
_CH2 = _EPT2_PAD // _CHUNK
_N_PAD = 10240
_RPT = _N_PAD // _NT
_RB = _RPT // _CHUNK

_BLK = 1000

_mesh = plsc.VectorSubcoreMesh(core_axis_name="c", subcore_axis_name="s")

_NO_LAYOUT_CP = pltpu.CompilerParams()
if "needs_layout_passes" in pltpu.CompilerParams.__dataclass_fields__:
  _NO_LAYOUT_CP = dataclasses.replace(_NO_LAYOUT_CP, needs_layout_passes=False)


def _make_agg(ch, nh):
  ch_h = ch // nh
  assert ch_h % _NBUF == 0

  @functools.partial(
      pl.kernel,
      out_type=jax.ShapeDtypeStruct((_NC, _N_PAD, 128), jnp.float32),
      mesh=_mesh,
      scratch_types=(
          [pltpu.VMEM((ch_h, _CHUNK), jnp.int32),
           pltpu.VMEM((ch_h, _CHUNK), jnp.int32)]
          + [pltpu.VMEM((_CHUNK, 128), jnp.float32)] * _NBUF
          + [pltpu.SemaphoreType.DMA] * (2 * _NBUF)
          + [pltpu.VMEM_SHARED((_N_PAD, 128), jnp.float32)]
      ),
  )
  def agg(z_hbm, gsrc_hbm, dst_hbm, out_hbm, src_v, dst_v, *rest):
    gbufs = rest[:_NBUF]
    gsems = rest[_NBUF:2 * _NBUF]
    ssems = rest[2 * _NBUF:3 * _NBUF]
    acc = rest[3 * _NBUF]
    c = lax.axis_index("c")
    s = lax.axis_index("s")
    base = s * _RPT

    @pl.loop(0, _CHUNK)
    def _(r):
      @pl.loop(0, 8)
      def _(k):
        gbufs[0][r, pl.ds(k * 16, 16)] = jnp.zeros((16,), jnp.float32)

    @pl.loop(0, _RB)
    def _(b):
      pltpu.sync_copy(gbufs[0], acc.at[pl.ds(base + b * _CHUNK, _CHUNK)])

    plsc.subcore_barrier()

    for h in range(nh):
      pltpu.sync_copy(gsrc_hbm.at[c, s, pl.ds(h * ch_h, ch_h)], src_v)
      pltpu.sync_copy(dst_hbm.at[c, s, pl.ds(h * ch_h, ch_h)], dst_v)

      for b in range(_NBUF):
        pltpu.async_copy(z_hbm.at[src_v.at[b]], gbufs[b], gsems[b])

      @pl.loop(0, ch_h // _NBUF)
      def _(j0):
        for b in range(_NBUF):
          j = j0 * _NBUF + b
          pltpu.make_async_copy(z_hbm.at[src_v.at[j]], gbufs[b],
                                gsems[b]).wait()
          pltpu.async_copy(gbufs[b], acc.at[dst_v.at[j]], ssems[b], add=True)
          nxt = j + _NBUF

          @pl.when(nxt < ch_h)
          def _():
            pltpu.make_async_copy(gbufs[b], acc.at[dst_v.at[j]],
                                  ssems[b]).wait()
            pltpu.async_copy(z_hbm.at[src_v.at[nxt]], gbufs[b], gsems[b])

      for b in range(_NBUF):
        j = ch_h - _NBUF + b
        pltpu.make_async_copy(gbufs[b], acc.at[dst_v.at[j]], ssems[b]).wait()

    plsc.subcore_barrier()
    pltpu.sync_copy(acc.at[pl.ds(base, _RPT)],
                    out_hbm.at[c, pl.ds(base, _RPT)])

  return agg


_HROWS = 384
_NPP = _HROWS * 128 // 16
_NPASS = 4
_AROWS = _N_PAD * 16 // 128
_ARPT = _AROWS // _NT
_MERGE = [3, 3, 3, 1]


@functools.partial(
    pl.kernel,
    out_type=jax.ShapeDtypeStruct((_NC, _N_PAD, 128), jnp.float32),
    mesh=_mesh,
    scratch_types=[
        pltpu.VMEM((_CHD * _CHUNK,), jnp.int32),
        pltpu.VMEM((_HROWS, 128), jnp.float32),
        pltpu.VMEM((_CHUNK, 128), jnp.float32),
        pltpu.VMEM((_ARPT, 128), jnp.float32),
        pltpu.VMEM((_AROWS // 128, 128), jnp.int32),
        pltpu.VMEM_SHARED((_AROWS, 128), jnp.float32),
    ],
    compiler_params=_NO_LAYOUT_CP,
)
def _deg_kernel(idx_hbm, zeros_hbm, idxm_hbm, out_hbm,
                idx_v, hist, zobuf, aslice, idxm, acc):
  c = lax.axis_index("c")
  s = lax.axis_index("s")
  lane = lax.iota(jnp.int32, 16)
  ones = jnp.ones((16,), jnp.float32)

  pltpu.sync_copy(zeros_hbm, zobuf)
  pltpu.sync_copy(idxm_hbm, idxm)
  pltpu.sync_copy(idx_hbm.at[c, s], idx_v)
  pltpu.sync_copy(zobuf.at[pl.ds(0, _ARPT)], acc.at[pl.ds(s * _ARPT, _ARPT)])
  plsc.subcore_barrier()

  for p in range(_NPASS):
    @pl.loop(0, _HROWS // _CHUNK)
    def _(b):
      pltpu.sync_copy(zeros_hbm, hist.at[pl.ds(b * _CHUNK, _CHUNK)])

    @pl.loop(0, (_CHD * _CHUNK) // 16)
    def _(i):
      vec = idx_v[pl.ds(i * 16, 16)]
      rel = vec - p * _NPP
      msk = (rel >= 0) & (rel < _NPP)
      relc = jnp.minimum(jnp.maximum(rel, 0), _NPP - 1)
      flat = relc * 16 + lane
      plsc.addupdate_scatter(
          hist, [lax.shift_right_logical(flat, 7), flat & 127], ones,
          mask=msk)

    for r3 in range(_MERGE[p]):
      pltpu.sync_copy(hist.at[pl.ds(r3 * 128, 128)],
                      acc.at[idxm.at[p * 3 + r3]], add=True)

  plsc.subcore_barrier()

  pltpu.sync_copy(acc.at[pl.ds(s * _ARPT, _ARPT)], aslice)
  base = s * _RPT

  @pl.loop(0, _RPT // _CHUNK)
  def _(sub):
    @pl.loop(0, _CHUNK // 16)
    def _(g):
      def _add(k, t):
        pos = sub * _CHUNK * 16 + g * 256 + lane * 16 + k
        return t + plsc.load_gather(
            aslice, [lax.shift_right_logical(pos, 7), pos & 127])

      tot = lax.fori_loop(0, 16, _add, jnp.zeros((16,), jnp.float32))
      plsc.store_scatter(zobuf, [g * 16 + lane, lane * 0], tot)

    pltpu.sync_copy(zobuf, out_hbm.at[c, pl.ds(base + sub * _CHUNK, _CHUNK)])


def _tc_mm1(features, W1):
  def body(x_ref, w_ref, out_ref):
    z = jnp.dot(x_ref[...], w_ref[...], preferred_element_type=jnp.float32)
    out_ref[0] = z[:, :128]
    out_ref[1] = z[:, 128:]

  return pl.pallas_call(
      body,
      grid=(_N // _BLK,),
      in_specs=[
          pl.BlockSpec((_BLK, _D_IN), lambda i: (i, 0)),
          pl.BlockSpec((_D_IN, _D_H), lambda i: (0, 0)),
      ],
      out_specs=pl.BlockSpec((2, _BLK, 128), lambda i: (0, i, 0)),
      out_shape=jax.ShapeDtypeStruct((2, _N, 128), jnp.float32),
  )(features, W1)


def _tc_scale1(zt, degs):
  def body(z_ref, deg_ref, out_ref):
    onorm = lax.rsqrt(jnp.maximum(deg_ref[0, :, 0:1], 1.0))
    out_ref[0] = z_ref[0] * onorm
    out_ref[1] = z_ref[1] * onorm

  return pl.pallas_call(
      body,
      grid=(_N // _BLK,),
      in_specs=[
          pl.BlockSpec((2, _BLK, 128), lambda i: (0, i, 0)),
          pl.BlockSpec((1, _BLK, 128), lambda i: (0, i, 0)),
      ],
      out_specs=pl.BlockSpec((2, _BLK, 128), lambda i: (0, i, 0)),
      out_shape=jax.ShapeDtypeStruct((2, _N, 128), jnp.float32),
  )(zt, degs)


def _tc_layer2(agg1, degs, b1, W2):
  def body(agg_ref, deg_ref, b1_ref, w_ref, out_ref):
    inorm = lax.rsqrt(jnp.maximum(deg_ref[1, :, 0:1], 1.0))
    onorm = lax.rsqrt(jnp.maximum(deg_ref[0, :, 0:1], 1.0))
    h = jnp.concatenate([agg_ref[0], agg_ref[1]], axis=1)
    h = h * inorm + b1_ref[...][None, :]
    h = jnp.maximum(h, 0.0)
    z = jnp.dot(h, w_ref[...], preferred_element_type=jnp.float32)
    z = z * onorm
    out_ref[...] = z

  return pl.pallas_call(
      body,
      grid=(_N // _BLK,),
      in_specs=[
          pl.BlockSpec((2, _BLK, 128), lambda i: (0, i, 0)),
          pl.BlockSpec((2, _BLK, 128), lambda i: (0, i, 0)),
          pl.BlockSpec((_D_H,), lambda i: (0,)),
          pl.BlockSpec((_D_H, _N_CLS), lambda i: (0, 0)),
      ],
      out_specs=pl.BlockSpec((_BLK, _N_CLS), lambda i: (i, 0)),
      out_shape=jax.ShapeDtypeStruct((_N, _N_CLS), jnp.float32),
  )(agg1, degs, b1, W2)


def _tc_finish(agg2, degs, b2):
  def body(agg_ref, deg_ref, b2_ref, out_ref):
    inorm = lax.rsqrt(jnp.maximum(deg_ref[1, :, 0:1], 1.0))
    h = agg_ref[0] + agg_ref[1]
    out_ref[...] = h * inorm + b2_ref[...][None, :]

  return pl.pallas_call(
      body,
      grid=(_N // _BLK,),
      in_specs=[
          pl.BlockSpec((2, _BLK, 128), lambda i: (0, i, 0)),
          pl.BlockSpec((2, _BLK, 128), lambda i: (0, i, 0)),
          pl.BlockSpec((_N_CLS,), lambda i: (0,)),
      ],
      out_specs=pl.BlockSpec((_BLK, _N_CLS), lambda i: (i, 0)),
      out_shape=jax.ShapeDtypeStruct((_N, _N_CLS), jnp.float32),
  )(agg2, degs, b2)


_agg_l1 = _make_agg(_CH, 2)
_agg_l2 = _make_agg(_CH2, 1)


def _pad_idx(idx_t, pad_vals, chunk):
  lead = idx_t.shape[:-1]
  padded = jnp.concatenate(
      [idx_t, jnp.broadcast_to(pad_vals, lead + (pad_vals.shape[-1],))],
      axis=-1)
  return padded.reshape(lead + (-1, chunk))


def kernel(features, edge_index, W1, b1, W2, b2):
  n = features.shape[0]
  src = edge_index[0].astype(jnp.int32)
  dst = edge_index[1].astype(jnp.int32)

  pad1 = _EPT_PAD - _EPT
  pad2 = _EPT2_PAD - _EPT2
  pmax = max(pad1, pad2)
  junk = _N + (jnp.arange(pmax, dtype=jnp.int32) % (_N_PAD - _N))
  valid = (jnp.arange(pmax, dtype=jnp.int32) * 37) % _N
  junk1, junk2 = junk[:pad1], junk[:pad2]
  valid1, valid2 = valid[:pad1], valid[:pad2]

  src_1 = _pad_idx(src.reshape(_NT, _EPT), valid1, _CHUNK)
  dst_1 = _pad_idx(dst.reshape(_NT, _EPT), junk1, _CHUNK)
  gsrc1 = jnp.stack([src_1, src_1 + n])
  dst1 = jnp.stack([dst_1, dst_1])

  src_2 = _pad_idx(src.reshape(_NC, _NT, _EPT2), valid2, _CHUNK)
  dst2 = _pad_idx(dst.reshape(_NC, _NT, _EPT2), junk2, _CHUNK)

  deg_idx = jnp.stack([
      _pad_idx(src.reshape(_NT, _EPT), junk1, _CHUNK),
      _pad_idx(dst.reshape(_NT, _EPT), junk1, _CHUNK),
  ]).reshape(2, _NT, _CHD * _CHUNK)

  zeros_c = jnp.zeros((_CHUNK, 128), jnp.float32)
  idxm_c = jnp.arange(_AROWS, dtype=jnp.int32).reshape(_AROWS // 128, 128)
  degs = _deg_kernel(deg_idx, zeros_c, idxm_c)
  zt = _tc_mm1(features, W1)
  z1 = _tc_scale1(zt, degs)
  agg1 = _agg_l1(z1.reshape(2 * n, 128), gsrc1, dst1)
  z2 = _tc_layer2(agg1, degs, b1, W2)
  agg2 = _agg_l2(z2, src_2, dst2)
  out = _tc_finish(agg2, degs, b2)
  return out


# speedup vs baseline: 1.0037x; 1.0037x over previous
import dataclasses
import functools

import jax
import jax.numpy as jnp
from jax import lax
from jax.experimental import pallas as pl
from jax.experimental.pallas import tpu as pltpu
from jax.experimental.pallas import tpu_sc as plsc

_N = 10000
_E = 160000
_D_IN = 256
_D_H = 256
_N_CLS = 128

_NT = 16
_NC = 2
_CHUNK = 128
_NBUF = 2
_EPT = _E // _NT
_EPT_PAD = 10240
_CH = _EPT_PAD // _CHUNK
_CHD = _EPT_PAD // _CHUNK
_EPT2 = _E // (_NC * _NT)
_EPT2_PAD = 5120
_CH2 = _EPT2_PAD // _CHUNK
_N_PAD = 10240
_RPT = _N_PAD // _NT
_RB = _RPT // _CHUNK

_BLK = 1000

_mesh = plsc.VectorSubcoreMesh(core_axis_name="c", subcore_axis_name="s")

_NO_LAYOUT_CP = pltpu.CompilerParams()
if "needs_layout_passes" in pltpu.CompilerParams.__dataclass_fields__:
  _NO_LAYOUT_CP = dataclasses.replace(_NO_LAYOUT_CP, needs_layout_passes=False)


def _make_agg(ch, nh):
  ch_h = ch // nh
  assert ch_h % _NBUF == 0

  @functools.partial(
      pl.kernel,
      out_type=jax.ShapeDtypeStruct((_NC, _N_PAD, 128), jnp.float32),
      mesh=_mesh,
      scratch_types=(
          [pltpu.VMEM((ch_h, _CHUNK), jnp.int32),
           pltpu.VMEM((ch_h, _CHUNK), jnp.int32)]
          + [pltpu.VMEM((_CHUNK, 128), jnp.float32)] * _NBUF
          + [pltpu.SemaphoreType.DMA] * (2 * _NBUF)
          + [pltpu.VMEM_SHARED((_N_PAD, 128), jnp.float32)]
      ),
  )
  def agg(z_hbm, gsrc_hbm, dst_hbm, out_hbm, src_v, dst_v, *rest):
    gbufs = rest[:_NBUF]
    gsems = rest[_NBUF:2 * _NBUF]
    ssems = rest[2 * _NBUF:3 * _NBUF]
    acc = rest[3 * _NBUF]
    c = lax.axis_index("c")
    s = lax.axis_index("s")
    base = s * _RPT

    @pl.loop(0, _CHUNK)
    def _(r):
      @pl.loop(0, 8)
      def _(k):
        gbufs[0][r, pl.ds(k * 16, 16)] = jnp.zeros((16,), jnp.float32)

    @pl.loop(0, _RB)
    def _(b):
      pltpu.sync_copy(gbufs[0], acc.at[pl.ds(base + b * _CHUNK, _CHUNK)])

    plsc.subcore_barrier()

    for h in range(nh):
      pltpu.sync_copy(gsrc_hbm.at[c, s, pl.ds(h * ch_h, ch_h)], src_v)
      pltpu.sync_copy(dst_hbm.at[c, s, pl.ds(h * ch_h, ch_h)], dst_v)

      for b in range(_NBUF):
        pltpu.async_copy(z_hbm.at[src_v.at[b]], gbufs[b], gsems[b])

      @pl.loop(0, ch_h // _NBUF)
      def _(j0):
        for b in range(_NBUF):
          j = j0 * _NBUF + b
          pltpu.make_async_copy(z_hbm.at[src_v.at[j]], gbufs[b],
                                gsems[b]).wait()
          pltpu.async_copy(gbufs[b], acc.at[dst_v.at[j]], ssems[b], add=True)
          nxt = j + _NBUF

          @pl.when(nxt < ch_h)
          def _():
            pltpu.make_async_copy(gbufs[b], acc.at[dst_v.at[j]],
                                  ssems[b]).wait()
            pltpu.async_copy(z_hbm.at[src_v.at[nxt]], gbufs[b], gsems[b])

      for b in range(_NBUF):
        j = ch_h - _NBUF + b
        pltpu.make_async_copy(gbufs[b], acc.at[dst_v.at[j]], ssems[b]).wait()

    plsc.subcore_barrier()
    pltpu.sync_copy(acc.at[pl.ds(base, _RPT)],
                    out_hbm.at[c, pl.ds(base, _RPT)])

  return agg


_HROWS = 384
_NPP = _HROWS * 128 // 16
_NPASS = 4
_AROWS = _N_PAD * 16 // 128
_ARPT = _AROWS // _NT
_MERGE = [3, 3, 3, 1]


@functools.partial(
    pl.kernel,
    out_type=jax.ShapeDtypeStruct((_NC, _N_PAD, 128), jnp.float32),
    mesh=_mesh,
    scratch_types=[
        pltpu.VMEM((_CHD * _CHUNK,), jnp.int32),
        pltpu.VMEM((_HROWS, 128), jnp.float32),
        pltpu.VMEM((_CHUNK, 128), jnp.float32),
        pltpu.VMEM((_ARPT, 128), jnp.float32),
        pltpu.VMEM((_AROWS // 128, 128), jnp.int32),
        pltpu.VMEM_SHARED((_AROWS, 128), jnp.float32),
    ],
    compiler_params=_NO_LAYOUT_CP,
)
def _deg_kernel(idx_hbm, zeros_hbm, idxm_hbm, out_hbm,
                idx_v, hist, zobuf, aslice, idxm, acc):
  c = lax.axis_index("c")
  s = lax.axis_index("s")
  lane = lax.iota(jnp.int32, 16)
  ones = jnp.ones((16,), jnp.float32)

  pltpu.sync_copy(zeros_hbm, zobuf)
  pltpu.sync_copy(idxm_hbm, idxm)
  pltpu.sync_copy(idx_hbm.at[c, s], idx_v)
  pltpu.sync_copy(zobuf.at[pl.ds(0, _ARPT)], acc.at[pl.ds(s * _ARPT, _ARPT)])
  plsc.subcore_barrier()

  for p in range(_NPASS):
    @pl.loop(0, _HROWS // _CHUNK)
    def _(b):
      pltpu.sync_copy(zeros_hbm, hist.at[pl.ds(b * _CHUNK, _CHUNK)])

    @pl.loop(0, (_CHD * _CHUNK) // 64)
    def _(i):
      for u in range(4):
        vec = idx_v[pl.ds(i * 64 + u * 16, 16)]
        rel = vec - p * _NPP
        msk = (rel >= 0) & (rel < _NPP)
        relc = jnp.minimum(jnp.maximum(rel, 0), _NPP - 1)
        flat = relc * 16 + lane
        plsc.addupdate_scatter(
            hist, [lax.shift_right_logical(flat, 7), flat & 127], ones,
            mask=msk)

    for r3 in range(_MERGE[p]):
      pltpu.sync_copy(hist.at[pl.ds(r3 * 128, 128)],
                      acc.at[idxm.at[p * 3 + r3]], add=True)

  plsc.subcore_barrier()

  pltpu.sync_copy(acc.at[pl.ds(s * _ARPT, _ARPT)], aslice)
  base = s * _RPT

  @pl.loop(0, _RPT // _CHUNK)
  def _(sub):
    @pl.loop(0, _CHUNK // 16)
    def _(g):
      tot = jnp.zeros((16,), jnp.float32)
      for k in range(16):
        pos = sub * _CHUNK * 16 + g * 256 + lane * 16 + k
        tot = tot + plsc.load_gather(
            aslice, [lax.shift_right_logical(pos, 7), pos & 127])
      plsc.store_scatter(zobuf, [g * 16 + lane, lane * 0], tot)

    pltpu.sync_copy(zobuf, out_hbm.at[c, pl.ds(base + sub * _CHUNK, _CHUNK)])


def _tc_mm1(features, W1):
  def body(x_ref, w_ref, out_ref):
    z = jnp.dot(x_ref[...], w_ref[...], preferred_element_type=jnp.float32)
    out_ref[0] = z[:, :128]
    out_ref[1] = z[:, 128:]

  return pl.pallas_call(
      body,
      grid=(_N // _BLK,),
      in_specs=[
          pl.BlockSpec((_BLK, _D_IN), lambda i: (i, 0)),
          pl.BlockSpec((_D_IN, _D_H), lambda i: (0, 0)),
      ],
      out_specs=pl.BlockSpec((2, _BLK, 128), lambda i: (0, i, 0)),
      out_shape=jax.ShapeDtypeStruct((2, _N, 128), jnp.float32),
  )(features, W1)


def _tc_scale1(zt, degs):
  def body(z_ref, deg_ref, out_ref):
    onorm = lax.rsqrt(jnp.maximum(deg_ref[0, :, 0:1], 1.0))
    out_ref[0] = z_ref[0] * onorm
    out_ref[1] = z_ref[1] * onorm

  return pl.pallas_call(
      body,
      grid=(_N // _BLK,),
      in_specs=[
          pl.BlockSpec((2, _BLK, 128), lambda i: (0, i, 0)),
          pl.BlockSpec((1, _BLK, 128), lambda i: (0, i, 0)),
      ],
      out_specs=pl.BlockSpec((2, _BLK, 128), lambda i: (0, i, 0)),
      out_shape=jax.ShapeDtypeStruct((2, _N, 128), jnp.float32),
  )(zt, degs)


def _tc_layer2(agg1, degs, b1, W2):
  def body(agg_ref, deg_ref, b1_ref, w_ref, out_ref):
    inorm = lax.rsqrt(jnp.maximum(deg_ref[1, :, 0:1], 1.0))
    onorm = lax.rsqrt(jnp.maximum(deg_ref[0, :, 0:1], 1.0))
    h = jnp.concatenate([agg_ref[0], agg_ref[1]], axis=1)
    h = h * inorm + b1_ref[...][None, :]
    h = jnp.maximum(h, 0.0)
    z = jnp.dot(h, w_ref[...], preferred_element_type=jnp.float32)
    z = z * onorm
    out_ref[...] = z

  return pl.pallas_call(
      body,
      grid=(_N // _BLK,),
      in_specs=[
          pl.BlockSpec((2, _BLK, 128), lambda i: (0, i, 0)),
          pl.BlockSpec((2, _BLK, 128), lambda i: (0, i, 0)),
          pl.BlockSpec((_D_H,), lambda i: (0,)),
          pl.BlockSpec((_D_H, _N_CLS), lambda i: (0, 0)),
      ],
      out_specs=pl.BlockSpec((_BLK, _N_CLS), lambda i: (i, 0)),
      out_shape=jax.ShapeDtypeStruct((_N, _N_CLS), jnp.float32),
  )(agg1, degs, b1, W2)


def _tc_finish(agg2, degs, b2):
  def body(agg_ref, deg_ref, b2_ref, out_ref):
    inorm = lax.rsqrt(jnp.maximum(deg_ref[1, :, 0:1], 1.0))
    h = agg_ref[0] + agg_ref[1]
    out_ref[...] = h * inorm + b2_ref[...][None, :]

  return pl.pallas_call(
      body,
      grid=(_N // _BLK,),
      in_specs=[
          pl.BlockSpec((2, _BLK, 128), lambda i: (0, i, 0)),
          pl.BlockSpec((2, _BLK, 128), lambda i: (0, i, 0)),
          pl.BlockSpec((_N_CLS,), lambda i: (0,)),
      ],
      out_specs=pl.BlockSpec((_BLK, _N_CLS), lambda i: (i, 0)),
      out_shape=jax.ShapeDtypeStruct((_N, _N_CLS), jnp.float32),
  )(agg2, degs, b2)


_agg_l1 = _make_agg(_CH, 2)
_agg_l2 = _make_agg(_CH2, 1)


def _pad_idx(idx_t, pad_vals, chunk):
  lead = idx_t.shape[:-1]
  padded = jnp.concatenate(
      [idx_t, jnp.broadcast_to(pad_vals, lead + (pad_vals.shape[-1],))],
      axis=-1)
  return padded.reshape(lead + (-1, chunk))


def kernel(features, edge_index, W1, b1, W2, b2):
  n = features.shape[0]
  src = edge_index[0].astype(jnp.int32)
  dst = edge_index[1].astype(jnp.int32)

  pad1 = _EPT_PAD - _EPT
  pad2 = _EPT2_PAD - _EPT2
  pmax = max(pad1, pad2)
  junk = _N + (jnp.arange(pmax, dtype=jnp.int32) % (_N_PAD - _N))
  valid = (jnp.arange(pmax, dtype=jnp.int32) * 37) % _N
  junk1, junk2 = junk[:pad1], junk[:pad2]
  valid1, valid2 = valid[:pad1], valid[:pad2]

  src_1 = _pad_idx(src.reshape(_NT, _EPT), valid1, _CHUNK)
  dst_1 = _pad_idx(dst.reshape(_NT, _EPT), junk1, _CHUNK)
  gsrc1 = jnp.stack([src_1, src_1 + n])
  dst1 = jnp.stack([dst_1, dst_1])

  src_2 = _pad_idx(src.reshape(_NC, _NT, _EPT2), valid2, _CHUNK)
  dst2 = _pad_idx(dst.reshape(_NC, _NT, _EPT2), junk2, _CHUNK)

  deg_idx = jnp.stack([
      _pad_idx(src.reshape(_NT, _EPT), junk1, _CHUNK),
      _pad_idx(dst.reshape(_NT, _EPT), junk1, _CHUNK),
  ]).reshape(2, _NT, _CHD * _CHUNK)

  zeros_c = jnp.zeros((_CHUNK, 128), jnp.float32)
  idxm_c = jnp.arange(_AROWS, dtype=jnp.int32).reshape(_AROWS // 128, 128)
  degs = _deg_kernel(deg_idx, zeros_c, idxm_c)
  zt = _tc_mm1(features, W1)
  z1 = _tc_scale1(zt, degs)
  agg1 = _agg_l1(z1.reshape(2 * n, 128), gsrc1, dst1)
  z2 = _tc_layer2(agg1, degs, b1, W2)
  agg2 = _agg_l2(z2, src_2, dst2)
  out = _tc_finish(agg2, degs, b2)
  return out


# speedup vs baseline: 1.0586x; 1.0547x over previous
import functools

import jax
import jax.numpy as jnp
from jax import lax
from jax.experimental import pallas as pl
from jax.experimental.pallas import tpu as pltpu
from jax.experimental.pallas import tpu_sc as plsc

_N = 10000
_E = 160000
_D_IN = 256
_D_H = 256
_N_CLS = 128

_NT = 16
_NC = 2
_CHUNK = 128
_NBUF = 2
_EPT = _E // _NT
_EPT_PAD = 10240
_CH = _EPT_PAD // _CHUNK
_CHD = _EPT_PAD // _CHUNK
_EPT2 = _E // (_NC * _NT)
_EPT2_PAD = 5120
_CH2 = _EPT2_PAD // _CHUNK
_N_PAD = 10240
_RPT = _N_PAD // _NT
_RB = _RPT // _CHUNK

_BLK = 1000

_mesh = plsc.VectorSubcoreMesh(core_axis_name="c", subcore_axis_name="s")


def _make_agg(ch, nh):
  ch_h = ch // nh
  assert ch_h % _NBUF == 0

  @functools.partial(
      pl.kernel,
      out_type=jax.ShapeDtypeStruct((_NC, _N_PAD, 128), jnp.float32),
      mesh=_mesh,
      scratch_types=(
          [pltpu.VMEM((ch_h, _CHUNK), jnp.int32),
           pltpu.VMEM((ch_h, _CHUNK), jnp.int32)]
          + [pltpu.VMEM((_CHUNK, 128), jnp.float32)] * _NBUF
          + [pltpu.SemaphoreType.DMA] * (2 * _NBUF)
          + [pltpu.VMEM_SHARED((_N_PAD, 128), jnp.float32)]
      ),
  )
  def agg(z_hbm, gsrc_hbm, dst_hbm, out_hbm, src_v, dst_v, *rest):
    gbufs = rest[:_NBUF]
    gsems = rest[_NBUF:2 * _NBUF]
    ssems = rest[2 * _NBUF:3 * _NBUF]
    acc = rest[3 * _NBUF]
    c = lax.axis_index("c")
    s = lax.axis_index("s")
    base = s * _RPT

    @pl.loop(0, _CHUNK)
    def _(r):
      @pl.loop(0, 8)
      def _(k):
        gbufs[0][r, pl.ds(k * 16, 16)] = jnp.zeros((16,), jnp.float32)

    @pl.loop(0, _RB)
    def _(b):
      pltpu.sync_copy(gbufs[0], acc.at[pl.ds(base + b * _CHUNK, _CHUNK)])

    plsc.subcore_barrier()

    for h in range(nh):
      pltpu.sync_copy(gsrc_hbm.at[c, s, pl.ds(h * ch_h, ch_h)], src_v)
      pltpu.sync_copy(dst_hbm.at[c, s, pl.ds(h * ch_h, ch_h)], dst_v)

      for b in range(_NBUF):
        pltpu.async_copy(z_hbm.at[src_v.at[b]], gbufs[b], gsems[b])

      @pl.loop(0, ch_h // _NBUF)
      def _(j0):
        for b in range(_NBUF):
          j = j0 * _NBUF + b
          pltpu.make_async_copy(z_hbm.at[src_v.at[j]], gbufs[b],
                                gsems[b]).wait()
          pltpu.async_copy(gbufs[b], acc.at[dst_v.at[j]], ssems[b], add=True)
          nxt = j + _NBUF

          @pl.when(nxt < ch_h)
          def _():
            pltpu.make_async_copy(gbufs[b], acc.at[dst_v.at[j]],
                                  ssems[b]).wait()
            pltpu.async_copy(z_hbm.at[src_v.at[nxt]], gbufs[b], gsems[b])

      for b in range(_NBUF):
        j = ch_h - _NBUF + b
        pltpu.make_async_copy(gbufs[b], acc.at[dst_v.at[j]], ssems[b]).wait()

    plsc.subcore_barrier()
    pltpu.sync_copy(acc.at[pl.ds(base, _RPT)],
                    out_hbm.at[c, pl.ds(base, _RPT)])

  return agg


@functools.partial(
    pl.kernel,
    out_type=jax.ShapeDtypeStruct((_NC, _N_PAD, 128), jnp.float32),
    mesh=_mesh,
    scratch_types=[
        pltpu.VMEM((_CHD, _CHUNK), jnp.int32),
        pltpu.VMEM((_CHUNK, 128), jnp.float32),
        pltpu.SemaphoreType.DMA,
        pltpu.VMEM_SHARED((_N_PAD, 128), jnp.float32),
    ],
)
def _deg_kernel(idx_hbm, out_hbm, idx_v, buf, dsem, acc):
  c = lax.axis_index("c")
  s = lax.axis_index("s")
  base = s * _RPT

  @pl.loop(0, _CHUNK)
  def _(r):
    @pl.loop(0, 8)
    def _(k):
      buf[r, pl.ds(k * 16, 16)] = jnp.zeros((16,), jnp.float32)

  @pl.loop(0, _RB)
  def _(b):
    pltpu.sync_copy(buf, acc.at[pl.ds(base + b * _CHUNK, _CHUNK)])

  plsc.subcore_barrier()

  @pl.loop(0, _CHUNK)
  def _(r):
    @pl.loop(0, 8)
    def _(k):
      buf[r, pl.ds(k * 16, 16)] = jnp.ones((16,), jnp.float32)

  pltpu.sync_copy(idx_hbm.at[c, s], idx_v)

  @pl.loop(0, _CHD)
  def _(j):
    pltpu.async_copy(buf, acc.at[idx_v.at[j]], dsem, add=True)

  @pl.loop(0, _CHD)
  def _(j):
    pltpu.make_async_copy(buf, acc.at[idx_v.at[j]], dsem).wait()

  plsc.subcore_barrier()
  pltpu.sync_copy(acc.at[pl.ds(base, _RPT)],
                  out_hbm.at[c, pl.ds(base, _RPT)])


def _tc_mm1(features, W1):
  def body(x_ref, w_ref, out_ref):
    z = jnp.dot(x_ref[...], w_ref[...], preferred_element_type=jnp.float32)
    out_ref[0] = z[:, :128]
    out_ref[1] = z[:, 128:]

  return pl.pallas_call(
      body,
      grid=(_N // _BLK,),
      in_specs=[
          pl.BlockSpec((_BLK, _D_IN), lambda i: (i, 0)),
          pl.BlockSpec((_D_IN, _D_H), lambda i: (0, 0)),
      ],
      out_specs=pl.BlockSpec((2, _BLK, 128), lambda i: (0, i, 0)),
      out_shape=jax.ShapeDtypeStruct((2, _N, 128), jnp.float32),
  )(features, W1)


def _tc_scale1(zt, degs):
  def body(z_ref, deg_ref, out_ref):
    onorm = lax.rsqrt(jnp.maximum(deg_ref[0, :, 0:1], 1.0))
    out_ref[0] = z_ref[0] * onorm
    out_ref[1] = z_ref[1] * onorm

  return pl.pallas_call(
      body,
      grid=(_N // _BLK,),
      in_specs=[
          pl.BlockSpec((2, _BLK, 128), lambda i: (0, i, 0)),
          pl.BlockSpec((1, _BLK, 128), lambda i: (0, i, 0)),
      ],
      out_specs=pl.BlockSpec((2, _BLK, 128), lambda i: (0, i, 0)),
      out_shape=jax.ShapeDtypeStruct((2, _N, 128), jnp.float32),
  )(zt, degs)


def _tc_layer2(agg1, degs, b1, W2):
  def body(agg_ref, deg_ref, b1_ref, w_ref, out_ref):
    inorm = lax.rsqrt(jnp.maximum(deg_ref[1, :, 0:1], 1.0))
    onorm = lax.rsqrt(jnp.maximum(deg_ref[0, :, 0:1], 1.0))
    h = jnp.concatenate([agg_ref[0], agg_ref[1]], axis=1)
    h = h * inorm + b1_ref[...][None, :]
    h = jnp.maximum(h, 0.0)
    z = jnp.dot(h, w_ref[...], preferred_element_type=jnp.float32)
    z = z * onorm
    out_ref[...] = z

  return pl.pallas_call(
      body,
      grid=(_N // _BLK,),
      in_specs=[
          pl.BlockSpec((2, _BLK, 128), lambda i: (0, i, 0)),
          pl.BlockSpec((2, _BLK, 128), lambda i: (0, i, 0)),
          pl.BlockSpec((_D_H,), lambda i: (0,)),
          pl.BlockSpec((_D_H, _N_CLS), lambda i: (0, 0)),
      ],
      out_specs=pl.BlockSpec((_BLK, _N_CLS), lambda i: (i, 0)),
      out_shape=jax.ShapeDtypeStruct((_N, _N_CLS), jnp.float32),
  )(agg1, degs, b1, W2)


def _tc_finish(agg2, degs, b2):
  def body(agg_ref, deg_ref, b2_ref, out_ref):
    inorm = lax.rsqrt(jnp.maximum(deg_ref[1, :, 0:1], 1.0))
    h = agg_ref[0] + agg_ref[1]
    out_ref[...] = h * inorm + b2_ref[...][None, :]

  return pl.pallas_call(
      body,
      grid=(_N // _BLK,),
      in_specs=[
          pl.BlockSpec((2, _BLK, 128), lambda i: (0, i, 0)),
          pl.BlockSpec((2, _BLK, 128), lambda i: (0, i, 0)),
          pl.BlockSpec((_N_CLS,), lambda i: (0,)),
      ],
      out_specs=pl.BlockSpec((_BLK, _N_CLS), lambda i: (i, 0)),
      out_shape=jax.ShapeDtypeStruct((_N, _N_CLS), jnp.float32),
  )(agg2, degs, b2)


_agg_l1 = _make_agg(_CH, 2)
_agg_l2 = _make_agg(_CH2, 1)


def _pad_idx(idx_t, pad_vals, chunk):
  lead = idx_t.shape[:-1]
  padded = jnp.concatenate(
      [idx_t, jnp.broadcast_to(pad_vals, lead + (pad_vals.shape[-1],))],
      axis=-1)
  return padded.reshape(lead + (-1, chunk))


def kernel(features, edge_index, W1, b1, W2, b2):
  n = features.shape[0]
  src = edge_index[0].astype(jnp.int32)
  dst = edge_index[1].astype(jnp.int32)

  pad1 = _EPT_PAD - _EPT
  pad2 = _EPT2_PAD - _EPT2
  pmax = max(pad1, pad2)
  junk = _N + (jnp.arange(pmax, dtype=jnp.int32) % (_N_PAD - _N))
  valid = (jnp.arange(pmax, dtype=jnp.int32) * 37) % _N
  junk1, junk2 = junk[:pad1], junk[:pad2]
  valid1, valid2 = valid[:pad1], valid[:pad2]

  src_1 = _pad_idx(src.reshape(_NT, _EPT), valid1, _CHUNK)
  dst_1 = _pad_idx(dst.reshape(_NT, _EPT), junk1, _CHUNK)
  gsrc1 = jnp.stack([src_1, src_1 + n])
  dst1 = jnp.stack([dst_1, dst_1])

  src_2 = _pad_idx(src.reshape(_NC, _NT, _EPT2), valid2, _CHUNK)
  dst2 = _pad_idx(dst.reshape(_NC, _NT, _EPT2), junk2, _CHUNK)

  deg_idx = jnp.stack([
      _pad_idx(src.reshape(_NT, _EPT), junk1, _CHUNK),
      _pad_idx(dst.reshape(_NT, _EPT), junk1, _CHUNK),
  ])

  degs = _deg_kernel(deg_idx)
  zt = _tc_mm1(features, W1)
  z1 = _tc_scale1(zt, degs)
  agg1 = _agg_l1(z1.reshape(2 * n, 128), gsrc1, dst1)
  z2 = _tc_layer2(agg1, degs, b1, W2)
  agg2 = _agg_l2(z2, src_2, dst2)
  out = _tc_finish(agg2, degs, b2)
  return out
